# baseline (speedup 1.0000x reference)
import numpy as np
import jax
import jax.numpy as jnp
from jax.experimental import pallas as pl
from jax.experimental.pallas import tpu as pltpu

N = 1024
D = 128
HEADS = 8
HEAD_DIM = 16
K = 10
KP = 16
KM_ITERS = 20
TEMP = 0.5

_EXP_1_OVER_T = np.float32(np.exp(np.float32(1.0 / TEMP)))


def _dotT(a, b, precision=None):
    return jax.lax.dot_general(a, b, (((1,), (1,)), ((), ())),
                               preferred_element_type=jnp.float32,
                               precision=precision)


def _fused(x_ref, w1_ref, b1_ref, w2_ref, b2_ref, wg_ref, asrc_ref,
           adst_ref, bg_ref, out_ref, loss_ref):
    X = x_ref[...]
    W1 = w1_ref[...]
    W2 = w2_ref[...]

    Hid = jnp.maximum(
        jnp.dot(X, W1, preferred_element_type=jnp.float32) + b1_ref[...], 0.0)
    Z = jnp.dot(Hid, W2, preferred_element_type=jnp.float32) + b2_ref[...]

    ones_n1 = jnp.ones((N, 1), jnp.float32)
    sq = jnp.sum(Z * Z, axis=1, keepdims=True)
    nrm = jnp.maximum(jnp.sqrt(sq), 1e-8)
    ZN = Z / nrm
    ZN2 = ZN + ZN
    S2 = _dotT(ZN2, ZN)
    pos2 = jnp.sum(ZN2 * ZN, axis=1, keepdims=True)
    Eexp = jnp.exp(S2)
    rs = jnp.dot(Eexp, ones_n1, preferred_element_type=jnp.float32)
    den = 2.0 * rs - _EXP_1_OVER_T
    nom = jnp.exp(pos2)
    loss_ref[...] = jnp.reshape(-jnp.mean(jnp.log(nom / den)), (1, 1))

    kiota = jax.lax.broadcasted_iota(jnp.int32, (N, KP), 1).astype(jnp.float32)

    ones_d1 = jnp.ones((D, 1), jnp.float32)

    def assign_of(cent):
        best = jnp.full((N, 1), jnp.inf, jnp.float32)
        bidx = jnp.zeros((N, 1), jnp.float32)
        for k in range(K):
            ck = cent[k:k + 1, :]
            diff = Z - ck
            d2k = jnp.dot(diff * diff, ones_d1,
                          preferred_element_type=jnp.float32,
                          precision=jax.lax.Precision.HIGHEST)
            take = d2k < best
            best = jnp.where(take, d2k, best)
            bidx = jnp.where(take, jnp.float32(k), bidx)
        return bidx

    def km_body(_, carry):
        cent, _ = carry
        bidx = assign_of(cent)
        onehot = (kiota == bidx).astype(jnp.float32)
        counts = jax.lax.dot_general(onehot, ones_n1, (((0,), (0,)), ((), ())),
                                     preferred_element_type=jnp.float32)
        centn = jax.lax.dot_general(onehot, Z, (((0,), (0,)), ((), ())),
                                    preferred_element_type=jnp.float32)
        return centn / jnp.maximum(counts, 1.0), bidx

    cent0 = Z[0:KP, :]
    _, bidx = jax.lax.fori_loop(0, KM_ITERS, km_body,
                                (cent0, jnp.zeros((N, 1), jnp.float32)))
    onehot = (kiota == bidx).astype(jnp.float32)
    maskf = _dotT(onehot, onehot)

    Hm = jnp.dot(Z, wg_ref[...], preferred_element_type=jnp.float32)
    a_dst = jnp.dot(Hm, adst_ref[...], preferred_element_type=jnp.float32)
    a_srcT = jax.lax.dot_general(asrc_ref[...], Hm, (((0,), (1,)), ((), ())),
                                 preferred_element_type=jnp.float32)

    bg = bg_ref[...]
    for h in range(HEADS):
        adh = a_dst[:, h:h + 1]
        ash = a_srcT[h:h + 1, :]
        Mh = jnp.maximum(jnp.max(adh, keepdims=True) +
                         jnp.max(ash, keepdims=True), 0.0)
        t1 = (adh - Mh) + ash
        t2 = (0.2 * adh - Mh) + 0.2 * ash
        p = maskf * jnp.exp(jnp.maximum(t1, t2))
        Hh = jnp.concatenate(
            [Hm[:, h * HEAD_DIM:(h + 1) * HEAD_DIM], ones_n1], axis=1)
        ohs = jnp.dot(p, Hh, preferred_element_type=jnp.float32)
        out_ref[:, h * HEAD_DIM:(h + 1) * HEAD_DIM] = (
            ohs[:, 0:HEAD_DIM] / ohs[:, HEAD_DIM:HEAD_DIM + 1] +
            bg[0:1, h * HEAD_DIM:(h + 1) * HEAD_DIM])


def kernel(x, W1, b1, W2, b2, Wg, att_src, att_dst, bg):
    bsz, npatch, nv, plen = x.shape
    X = x.reshape(bsz * npatch, nv * plen)
    eyeH = jnp.eye(HEADS, dtype=jnp.float32)
    Asrc = (att_src[:, :, None] * eyeH[:, None, :]).reshape(D, HEADS)
    Adst = (att_dst[:, :, None] * eyeH[:, None, :]).reshape(D, HEADS)

    out, loss = pl.pallas_call(
        _fused,
        out_shape=[
            jax.ShapeDtypeStruct((N, D), jnp.float32),
            jax.ShapeDtypeStruct((1, 1), jnp.float32),
        ],
    )(X, W1, b1.reshape(1, D), W2, b2.reshape(1, D), Wg, Asrc, Adst,
      bg.reshape(1, D))
    return out.reshape(bsz, npatch, nv, plen), loss.reshape(())


# speedup vs baseline: 2.2929x; 2.2929x over previous
import numpy as np
import jax
import jax.numpy as jnp
from jax.experimental import pallas as pl
from jax.experimental.pallas import tpu as pltpu

N = 1024
D = 128
HEADS = 8
HEAD_DIM = 16
K = 10
KP = 16
KM_ITERS = 20
TEMP = 0.5

_EXP_1_OVER_T = np.float32(np.exp(np.float32(1.0 / TEMP)))


def _dotT(a, b, precision=None):
    return jax.lax.dot_general(a, b, (((1,), (1,)), ((), ())),
                               preferred_element_type=jnp.float32,
                               precision=precision)


def _fused(x_ref, w1_ref, b1_ref, w2_ref, b2_ref, wg_ref, asrc_ref,
           adst_ref, bg_ref, out_ref, loss_ref):
    X = x_ref[...]
    W1 = w1_ref[...]
    W2 = w2_ref[...]

    Hid = jnp.maximum(
        jnp.dot(X, W1, preferred_element_type=jnp.float32) + b1_ref[...], 0.0)
    Z = jnp.dot(Hid, W2, preferred_element_type=jnp.float32) + b2_ref[...]

    ones_n1 = jnp.ones((N, 1), jnp.float32)
    sq = jnp.sum(Z * Z, axis=1, keepdims=True)
    nrm = jnp.maximum(jnp.sqrt(sq), 1e-8)
    ZN = Z / nrm
    ZN2 = ZN + ZN
    S2 = _dotT(ZN2, ZN)
    pos2 = jnp.sum(ZN2 * ZN, axis=1, keepdims=True)
    Eexp = jnp.exp(S2)
    rs = jnp.dot(Eexp, ones_n1, preferred_element_type=jnp.float32)
    den = 2.0 * rs - _EXP_1_OVER_T
    nom = jnp.exp(pos2)
    loss_ref[...] = jnp.reshape(-jnp.mean(jnp.log(nom / den)), (1, 1))

    def bf16_split2(x):
        hi = x.astype(jnp.bfloat16).astype(jnp.float32)
        lo = (x - hi).astype(jnp.bfloat16).astype(jnp.float32)
        return hi, lo

    def bf16_split3(x):
        hi = x.astype(jnp.bfloat16).astype(jnp.float32)
        r = x - hi
        mid = r.astype(jnp.bfloat16).astype(jnp.float32)
        lo = r - mid
        return hi, mid, lo

    kio_col = jax.lax.broadcasted_iota(jnp.int32, (KP, 1), 0).astype(jnp.float32)
    EYE = (jax.lax.broadcasted_iota(jnp.int32, (N, N), 0) ==
           jax.lax.broadcasted_iota(jnp.int32, (N, N), 1)).astype(jnp.float32)
    INF = jnp.float32(jnp.inf)
    Z_hi, Z_lo = bf16_split2(Z)
    Zcat = jnp.concatenate([Z_hi, Z_lo, Z_hi, Z_lo], axis=1)

    def assign_of(cent):
        c_hi, c_mid, c_lo = bf16_split3(cent)
        Ccat = jnp.concatenate([c_hi, c_hi, c_mid, c_mid], axis=1)
        GT = jax.lax.dot_general(Ccat, Zcat, (((1,), (1,)), ((), ())),
                                 preferred_element_type=jnp.float32)
        csq = jnp.sum(cent * cent, axis=1, keepdims=True)
        scoreT = csq - 2.0 * GT
        scoreT = jnp.where(kio_col < jnp.float32(K), scoreT, INF)
        b1 = jnp.min(scoreT, axis=0, keepdims=True)
        i1r = jnp.min(jnp.where(scoreT == b1, kio_col, jnp.float32(KP)),
                      axis=0, keepdims=True)
        score2 = jnp.where(kio_col == i1r, INF, scoreT)
        b2 = jnp.min(score2, axis=0, keepdims=True)
        i2r = jnp.min(jnp.where(score2 == b2, kio_col, jnp.float32(KP)),
                      axis=0, keepdims=True)
        oh1 = (kio_col == i1r).astype(jnp.float32)
        oh2 = (kio_col == i2r).astype(jnp.float32)
        Cstack = jnp.concatenate([c_hi, c_mid, c_lo], axis=0)
        oh1t = jnp.concatenate([oh1, oh1, oh1], axis=0)
        oh2t = jnp.concatenate([oh2, oh2, oh2], axis=0)
        c1 = jax.lax.dot_general(oh1t, Cstack, (((0,), (0,)), ((), ())),
                                 preferred_element_type=jnp.float32)
        c2 = jax.lax.dot_general(oh2t, Cstack, (((0,), (0,)), ((), ())),
                                 preferred_element_type=jnp.float32)
        dd1 = Z - c1
        e1 = jnp.sum(dd1 * dd1, axis=1, keepdims=True)
        dd2 = Z - c2
        e2 = jnp.sum(dd2 * dd2, axis=1, keepdims=True)
        i1c = jax.lax.dot_general(EYE, i1r, (((1,), (1,)), ((), ())),
                                  preferred_element_type=jnp.float32)
        i2c = jax.lax.dot_general(EYE, i2r, (((1,), (1,)), ((), ())),
                                  preferred_element_type=jnp.float32)
        take2 = (e2 < e1) | ((e2 == e1) & (i2c < i1c))
        bidx_col = jnp.where(take2, i2c, i1c)
        return jax.lax.dot_general(bidx_col, EYE, (((0,), (0,)), ((), ())),
                                   preferred_element_type=jnp.float32)

    def km_body(_, carry):
        cent, _ = carry
        bidx_row = assign_of(cent)
        ohT = (kio_col == bidx_row).astype(jnp.float32)
        counts = jax.lax.dot_general(ohT, ones_n1, (((1,), (0,)), ((), ())),
                                     preferred_element_type=jnp.float32)
        centn = jax.lax.dot_general(ohT, Z, (((1,), (0,)), ((), ())),
                                    preferred_element_type=jnp.float32)
        return centn / jnp.maximum(counts, 1.0), bidx_row

    cent0 = Z[0:KP, :]
    _, bidx_row = jax.lax.fori_loop(0, KM_ITERS, km_body,
                                    (cent0, jnp.zeros((1, N), jnp.float32)))
    ohT = (kio_col == bidx_row).astype(jnp.float32)
    maskf = jax.lax.dot_general(ohT, ohT, (((0,), (0,)), ((), ())),
                                preferred_element_type=jnp.float32)

    Hm = jnp.dot(Z, wg_ref[...], preferred_element_type=jnp.float32)
    a_dst = jnp.dot(Hm, adst_ref[...], preferred_element_type=jnp.float32)
    a_srcT = jax.lax.dot_general(asrc_ref[...], Hm, (((0,), (1,)), ((), ())),
                                 preferred_element_type=jnp.float32)

    bg = bg_ref[...]
    for h in range(HEADS):
        adh = a_dst[:, h:h + 1]
        ash = a_srcT[h:h + 1, :]
        Mh = jnp.maximum(jnp.max(adh, keepdims=True) +
                         jnp.max(ash, keepdims=True), 0.0)
        t1 = (adh - Mh) + ash
        t2 = (0.2 * adh - Mh) + 0.2 * ash
        p = maskf * jnp.exp(jnp.maximum(t1, t2))
        Hh = jnp.concatenate(
            [Hm[:, h * HEAD_DIM:(h + 1) * HEAD_DIM], ones_n1], axis=1)
        ohs = jnp.dot(p, Hh, preferred_element_type=jnp.float32)
        out_ref[:, h * HEAD_DIM:(h + 1) * HEAD_DIM] = (
            ohs[:, 0:HEAD_DIM] / ohs[:, HEAD_DIM:HEAD_DIM + 1] +
            bg[0:1, h * HEAD_DIM:(h + 1) * HEAD_DIM])


def kernel(x, W1, b1, W2, b2, Wg, att_src, att_dst, bg):
    bsz, npatch, nv, plen = x.shape
    X = x.reshape(bsz * npatch, nv * plen)
    eyeH = jnp.eye(HEADS, dtype=jnp.float32)
    Asrc = (att_src[:, :, None] * eyeH[:, None, :]).reshape(D, HEADS)
    Adst = (att_dst[:, :, None] * eyeH[:, None, :]).reshape(D, HEADS)

    out, loss = pl.pallas_call(
        _fused,
        out_shape=[
            jax.ShapeDtypeStruct((N, D), jnp.float32),
            jax.ShapeDtypeStruct((1, 1), jnp.float32),
        ],
    )(X, W1, b1.reshape(1, D), W2, b2.reshape(1, D), Wg, Asrc, Adst,
      bg.reshape(1, D))
    return out.reshape(bsz, npatch, nv, plen), loss.reshape(())
